# Initial kernel scaffold; baseline (speedup 1.0000x reference)
import jax
import jax.numpy as jnp
from jax.experimental import pallas as pl


def kernel(x, batch, gamma, beta):
    raise NotImplementedError("write your pallas kernel here")


# speedup vs baseline: 16.1401x; 16.1401x over previous
import jax
import jax.numpy as jnp
from jax.experimental import pallas as pl
from jax.experimental.pallas import tpu as pltpu

_N = 50000
_H = 256
_G = 64
_EPS = 1e-05
_B = 2000
_NB = _N // _B


def _stats_kernel(x_ref, b_ref, sum_ref, sq_ref, cnt_ref):
    i = pl.program_id(0)
    x = x_ref[...]
    b = b_ref[0, 0, :]
    gids = jax.lax.broadcasted_iota(jnp.int32, (_G, 1), 0)
    onehot_t = (gids == b[None, :]).astype(jnp.float32)
    s = jnp.dot(onehot_t, x, preferred_element_type=jnp.float32)
    q = jnp.dot(onehot_t, x * x, preferred_element_type=jnp.float32)
    c = jnp.sum(onehot_t, axis=1, keepdims=True)

    @pl.when(i == 0)
    def _():
        sum_ref[...] = s
        sq_ref[...] = q
        cnt_ref[...] = c

    @pl.when(i > 0)
    def _():
        sum_ref[...] += s
        sq_ref[...] += q
        cnt_ref[...] += c


def _norm_kernel(x_ref, b_ref, sum_ref, sq_ref, cnt_ref, gam_ref, bet_ref,
                 o_ref, a_ref, c_ref):
    i = pl.program_id(0)

    @pl.when(i == 0)
    def _():
        cnt = jnp.maximum(cnt_ref[...], 1.0)
        mean = sum_ref[...] / cnt
        var = jnp.maximum(sq_ref[...] / cnt - mean * mean, 0.0)
        inv = 1.0 / (jnp.sqrt(var + _EPS) + _EPS)
        gam = gam_ref[...]
        a_ref[...] = inv * gam
        c_ref[...] = bet_ref[...] - mean * inv * gam

    b = b_ref[0, 0, :]
    gids = jax.lax.broadcasted_iota(jnp.int32, (_G, 1), 0)
    onehot_t = (gids == b[None, :]).astype(jnp.float32)
    dn = (((0,), (0,)), ((), ()))
    a_n = jax.lax.dot_general(onehot_t, a_ref[...], dn,
                              preferred_element_type=jnp.float32)
    c_n = jax.lax.dot_general(onehot_t, c_ref[...], dn,
                              preferred_element_type=jnp.float32)
    o_ref[...] = x_ref[...] * a_n + c_n


def kernel(x, batch, gamma, beta):
    batch3 = batch.reshape(_NB, 1, _B)
    gamma2 = gamma.reshape(1, _H)
    beta2 = beta.reshape(1, _H)

    sums, sq, cnt = pl.pallas_call(
        _stats_kernel,
        grid=(_NB,),
        in_specs=[
            pl.BlockSpec((_B, _H), lambda i: (i, 0)),
            pl.BlockSpec((1, 1, _B), lambda i: (i, 0, 0)),
        ],
        out_specs=[
            pl.BlockSpec((_G, _H), lambda i: (0, 0)),
            pl.BlockSpec((_G, _H), lambda i: (0, 0)),
            pl.BlockSpec((_G, 1), lambda i: (0, 0)),
        ],
        out_shape=[
            jax.ShapeDtypeStruct((_G, _H), jnp.float32),
            jax.ShapeDtypeStruct((_G, _H), jnp.float32),
            jax.ShapeDtypeStruct((_G, 1), jnp.float32),
        ],
    )(x, batch3)

    out = pl.pallas_call(
        _norm_kernel,
        grid=(_NB,),
        in_specs=[
            pl.BlockSpec((_B, _H), lambda i: (i, 0)),
            pl.BlockSpec((1, 1, _B), lambda i: (i, 0, 0)),
            pl.BlockSpec((_G, _H), lambda i: (0, 0)),
            pl.BlockSpec((_G, _H), lambda i: (0, 0)),
            pl.BlockSpec((_G, 1), lambda i: (0, 0)),
            pl.BlockSpec((1, _H), lambda i: (0, 0)),
            pl.BlockSpec((1, _H), lambda i: (0, 0)),
        ],
        out_specs=pl.BlockSpec((_B, _H), lambda i: (i, 0)),
        out_shape=jax.ShapeDtypeStruct((_N, _H), jnp.float32),
        scratch_shapes=[
            pltpu.VMEM((_G, _H), jnp.float32),
            pltpu.VMEM((_G, _H), jnp.float32),
        ],
    )(x, batch3, sums, sq, cnt, gamma2, beta2)
    return out
